# baseline (speedup 1.0000x reference)
import jax
import jax.numpy as jnp
from jax import lax
from jax.experimental import pallas as pl
from jax.experimental.pallas import tpu as pltpu, tpu_sc as plsc

NUM_Q = 4
CB_DIM = 64
CB_SIZE = 1024
BETA = 0.25
B, H, W = 8, 32, 32
N = H * W
UNROLL = 4
NB = B // UNROLL
NTOK = B * N

SC_CORES = 2
SC_SUBCORES = 16
SC_WORKERS = SC_CORES * SC_SUBCORES
ROWS_PER_W = NTOK // SC_WORKERS


def _dist_block(xb, cb, cbnorm, iota_k):
    znorm = jnp.sum(xb * xb, axis=0, keepdims=True)
    dots = jax.lax.dot(cb, xb)
    scores = (znorm - 2.0 * dots) + cbnorm
    m = jnp.min(scores, axis=0, keepdims=True)
    idx = jnp.min(jnp.where(scores == m, iota_k, CB_SIZE), axis=0,
                  keepdims=True)
    return idx, jnp.sum(m)


def _dist_kernel(x_ref, cb_ref, idx_ref, loss_ref):
    b = pl.program_id(0)
    cb = cb_ref[0]
    cbnorm = jnp.sum(cb * cb, axis=1, keepdims=True)
    iota_k = jax.lax.broadcasted_iota(jnp.int32, (CB_SIZE, 1), 0)

    loss_c = None
    for s in range(UNROLL):
        idx, loss = _dist_block(x_ref[s, 0], cb, cbnorm, iota_k)
        idx_ref[0:1, s * N:(s + 1) * N] = idx
        loss_c = loss if loss_c is None else loss_c + loss

    @pl.when(b == 0)
    def _init():
        loss_ref[...] = jnp.full((1, 128), loss_c, jnp.float32)

    @pl.when(b > 0)
    def _acc():
        loss_ref[...] = loss_ref[...] + loss_c

    @pl.when(b == NB - 1)
    def _finalize():
        loss_ref[...] = loss_ref[...] * ((1.0 + BETA) / (B * N * CB_DIM))


def _gather_kernel(cb_hbm, idx_hbm, rows_hbm, hist_hbm,
                   idx_v, rows_v, ones_v, zeros_v, hist_sh, sem):
    cid = lax.axis_index("c")
    sid = lax.axis_index("s")
    wid = sid * SC_CORES + cid
    base = wid * ROWS_PER_W

    pltpu.sync_copy(idx_hbm.at[pl.ds(base, ROWS_PER_W)], idx_v)

    @pl.loop(0, ROWS_PER_W // 16)
    def _prep(j):
        ones_v[pl.ds(j * 16, 16)] = jnp.ones((16,), jnp.float32)

    @pl.loop(0, CB_SIZE // 16)
    def _zero(j):
        zeros_v[pl.ds(j * 16, 16)] = jnp.zeros((16,), jnp.float32)

    @pl.when(sid == 0)
    def _init_hist():
        pltpu.sync_copy(zeros_v, hist_sh)

    plsc.subcore_barrier()

    pltpu.async_copy(cb_hbm.at[idx_v], rows_v, sem).wait()
    pltpu.sync_copy(rows_v, rows_hbm.at[pl.ds(base, ROWS_PER_W)])

    pltpu.sync_copy(ones_v, hist_sh.at[idx_v], add=True)
    plsc.subcore_barrier()

    @pl.when(sid == 0)
    def _pub():
        pltpu.sync_copy(hist_sh, hist_hbm.at[cid])


def _perp_kernel(hist_ref, perp_ref):
    hist = hist_ref[:, 0] + hist_ref[:, 1]
    probs = hist * (1.0 / (B * N))
    plogp = probs * jnp.log(probs + 1e-10)
    ent = jnp.sum(plogp, axis=1, keepdims=True)
    perp_ref[...] = jnp.exp(-ent) * jnp.ones((NUM_Q, 128), jnp.float32)


def _make_sc_gather():
    return pl.kernel(
        _gather_kernel,
        out_type=[
            jax.ShapeDtypeStruct((NTOK, CB_DIM), jnp.float32),
            jax.ShapeDtypeStruct((SC_CORES, CB_SIZE), jnp.float32),
        ],
        mesh=plsc.VectorSubcoreMesh(core_axis_name="c", subcore_axis_name="s"),
        scratch_types=[
            pltpu.VMEM((ROWS_PER_W,), jnp.int32),
            pltpu.VMEM((ROWS_PER_W, CB_DIM), jnp.float32),
            pltpu.VMEM((ROWS_PER_W,), jnp.float32),
            pltpu.VMEM((CB_SIZE,), jnp.float32),
            pltpu.VMEM_SHARED((CB_SIZE,), jnp.float32),
            pltpu.SemaphoreType.DMA,
        ],
        compiler_params=pltpu.CompilerParams(use_tc_tiling_on_sc=False),
    )


@jax.jit
def kernel(x, codebooks):
    xr = x.reshape(B, NUM_Q, CB_DIM, N)
    sc_gather = _make_sc_gather()

    idx_l, loss_l, rows_l, hist_l = [], [], [], []
    for i in range(NUM_Q):
        idx, loss = pl.pallas_call(
            _dist_kernel,
            grid=(NB,),
            in_specs=[
                pl.BlockSpec((UNROLL, 1, CB_DIM, N),
                             lambda b, i=i: (b, i, 0, 0)),
                pl.BlockSpec((1, CB_SIZE, CB_DIM), lambda b, i=i: (i, 0, 0)),
            ],
            out_specs=[
                pl.BlockSpec((1, UNROLL * N), lambda b: (0, b)),
                pl.BlockSpec((1, 128), lambda b: (0, 0)),
            ],
            out_shape=[
                jax.ShapeDtypeStruct((1, NTOK), jnp.int32),
                jax.ShapeDtypeStruct((1, 128), jnp.float32),
            ],
        )(xr, codebooks)
        rows, hist2 = sc_gather(codebooks[i], idx.reshape(NTOK))
        idx_l.append(idx)
        loss_l.append(loss)
        rows_l.append(rows)
        hist_l.append(hist2)

    perp = pl.pallas_call(
        _perp_kernel,
        grid=(1,),
        in_specs=[pl.BlockSpec((NUM_Q, SC_CORES, CB_SIZE),
                               lambda _: (0, 0, 0))],
        out_specs=pl.BlockSpec((NUM_Q, 128), lambda _: (0, 0)),
        out_shape=jax.ShapeDtypeStruct((NUM_Q, 128), jnp.float32),
    )(jnp.stack(hist_l))

    quant = jnp.stack(rows_l)
    quantized_cat = quant.reshape(NUM_Q, B, N, CB_DIM).transpose(
        1, 0, 3, 2).reshape(B, NUM_Q * CB_DIM, H, W)
    indices_cat = jnp.stack(
        [ix.reshape(B, H, W) for ix in idx_l], axis=1)
    loss_cat = jnp.concatenate([l[0, :1] for l in loss_l])
    perplexity_cat = perp[:, 0]
    return (quantized_cat, indices_cat, loss_cat, perplexity_cat)


# speedup vs baseline: 1.1623x; 1.1623x over previous
import jax
import jax.numpy as jnp
from jax import lax
from jax.experimental import pallas as pl
from jax.experimental.pallas import tpu as pltpu, tpu_sc as plsc

NUM_Q = 4
CB_DIM = 64
CB_SIZE = 1024
BETA = 0.25
B, H, W = 8, 32, 32
N = H * W
UNROLL = 4
NB = B // UNROLL
NTOK = NUM_Q * B * N

SC_CORES = 2
SC_SUBCORES = 16
SC_WORKERS = SC_CORES * SC_SUBCORES
ROWS_PER_W = NTOK // SC_WORKERS
KFLAT = NUM_Q * CB_SIZE


def _dist_block(xb, cb, cbnorm, iota_k):
    znorm = jnp.sum(xb * xb, axis=0, keepdims=True)
    dots = jax.lax.dot(cb, xb)
    scores = (znorm - 2.0 * dots) + cbnorm
    m = jnp.min(scores, axis=0, keepdims=True)
    idx = jnp.min(jnp.where(scores == m, iota_k, CB_SIZE), axis=0,
                  keepdims=True)
    return idx, jnp.sum(m)


def _dist_kernel(x_ref, cb_ref, idx_ref, loss_ref):
    i = pl.program_id(0)
    b = pl.program_id(1)
    cb = cb_ref[0]
    cbnorm = jnp.sum(cb * cb, axis=1, keepdims=True)
    iota_k = jax.lax.broadcasted_iota(jnp.int32, (CB_SIZE, 1), 0)

    loss_c = None
    for s in range(UNROLL):
        idx, loss = _dist_block(x_ref[s, 0], cb, cbnorm, iota_k)
        idx_ref[0, 0:1, s * N:(s + 1) * N] = idx + i * CB_SIZE
        loss_c = loss if loss_c is None else loss_c + loss

    @pl.when(b == 0)
    def _init():
        loss_ref[0] = jnp.full((1, 128), loss_c, jnp.float32)

    @pl.when(b > 0)
    def _acc():
        loss_ref[0] = loss_ref[0] + loss_c

    @pl.when(b == NB - 1)
    def _finalize():
        loss_ref[0] = loss_ref[0] * ((1.0 + BETA) / (B * N * CB_DIM))


def _gather_kernel(cb_hbm, idx_hbm, rows_hbm, hist_hbm,
                   idx_v, rows_v, ones_v, zeros_v, hist_sh, sem):
    cid = lax.axis_index("c")
    sid = lax.axis_index("s")
    wid = sid * SC_CORES + cid
    base = wid * ROWS_PER_W

    pltpu.sync_copy(idx_hbm.at[pl.ds(base, ROWS_PER_W)], idx_v)

    @pl.loop(0, ROWS_PER_W // 16)
    def _prep(j):
        ones_v[pl.ds(j * 16, 16)] = jnp.ones((16,), jnp.float32)

    @pl.when(sid == 0)
    def _init_hist():
        @pl.loop(0, KFLAT // 16)
        def _zero(j):
            zeros_v[pl.ds(j * 16, 16)] = jnp.zeros((16,), jnp.float32)
        pltpu.sync_copy(zeros_v, hist_sh)

    plsc.subcore_barrier()

    pltpu.async_copy(cb_hbm.at[idx_v], rows_v, sem).wait()
    pltpu.sync_copy(rows_v, rows_hbm.at[pl.ds(base, ROWS_PER_W)])

    pltpu.sync_copy(ones_v, hist_sh.at[idx_v], add=True)
    plsc.subcore_barrier()

    @pl.when(sid == 0)
    def _pub():
        pltpu.sync_copy(hist_sh, hist_hbm.at[cid])


def _perp_kernel(hist_ref, perp_ref):
    hist = hist_ref[0] + hist_ref[1]
    probs = hist * (1.0 / (B * N))
    plogp = probs * jnp.log(probs + 1e-10)
    ent = jnp.sum(plogp, axis=1, keepdims=True)
    perp_ref[...] = jnp.exp(-ent) * jnp.ones((NUM_Q, 128), jnp.float32)


@jax.jit
def kernel(x, codebooks):
    xr = x.reshape(B, NUM_Q, CB_DIM, N)
    idx, loss = pl.pallas_call(
        _dist_kernel,
        grid=(NUM_Q, NB),
        in_specs=[
            pl.BlockSpec((UNROLL, 1, CB_DIM, N), lambda i, b: (b, i, 0, 0)),
            pl.BlockSpec((1, CB_SIZE, CB_DIM), lambda i, b: (i, 0, 0)),
        ],
        out_specs=[
            pl.BlockSpec((1, 1, UNROLL * N), lambda i, b: (i, 0, b)),
            pl.BlockSpec((1, 1, 128), lambda i, b: (i, 0, 0)),
        ],
        out_shape=[
            jax.ShapeDtypeStruct((NUM_Q, 1, B * N), jnp.int32),
            jax.ShapeDtypeStruct((NUM_Q, 1, 128), jnp.float32),
        ],
    )(xr, codebooks)

    sc_gather = pl.kernel(
        _gather_kernel,
        out_type=[
            jax.ShapeDtypeStruct((NTOK, CB_DIM), jnp.float32),
            jax.ShapeDtypeStruct((SC_CORES, KFLAT), jnp.float32),
        ],
        mesh=plsc.VectorSubcoreMesh(core_axis_name="c", subcore_axis_name="s"),
        scratch_types=[
            pltpu.VMEM((ROWS_PER_W,), jnp.int32),
            pltpu.VMEM((ROWS_PER_W, CB_DIM), jnp.float32),
            pltpu.VMEM((ROWS_PER_W,), jnp.float32),
            pltpu.VMEM((KFLAT,), jnp.float32),
            pltpu.VMEM_SHARED((KFLAT,), jnp.float32),
            pltpu.SemaphoreType.DMA,
        ],
        compiler_params=pltpu.CompilerParams(use_tc_tiling_on_sc=False),
    )
    rows, hist2 = sc_gather(codebooks.reshape(KFLAT, CB_DIM),
                            idx.reshape(NTOK))

    perp = pl.pallas_call(
        _perp_kernel,
        grid=(1,),
        in_specs=[pl.BlockSpec((SC_CORES, NUM_Q, CB_SIZE),
                               lambda _: (0, 0, 0))],
        out_specs=pl.BlockSpec((NUM_Q, 128), lambda _: (0, 0)),
        out_shape=jax.ShapeDtypeStruct((NUM_Q, 128), jnp.float32),
    )(hist2.reshape(SC_CORES, NUM_Q, CB_SIZE))

    quantized_cat = rows.reshape(NUM_Q, B, N, CB_DIM).transpose(
        1, 0, 3, 2).reshape(B, NUM_Q * CB_DIM, H, W)
    offs = (jnp.arange(NUM_Q, dtype=jnp.int32) * CB_SIZE)[None, :, None, None]
    indices_cat = idx.reshape(NUM_Q, B, H, W).transpose(1, 0, 2, 3) - offs
    loss_cat = loss[:, 0, 0]
    perplexity_cat = perp[:, 0]
    return (quantized_cat, indices_cat, loss_cat, perplexity_cat)


# speedup vs baseline: 1.2765x; 1.0983x over previous
import jax
import jax.numpy as jnp
from jax.experimental import pallas as pl

NUM_Q = 4
CB_DIM = 64
CB_SIZE = 1024
BETA = 0.25
B, H, W = 8, 32, 32
N = H * W
UNROLL = 8
NB = B // UNROLL


def _vq_block(xb, cb, cbnorm, iota_k):
    znorm = jnp.sum(xb * xb, axis=0, keepdims=True)
    dots = jax.lax.dot(cb, xb)
    scores = (znorm - 2.0 * dots) + cbnorm

    m = jnp.min(scores, axis=0, keepdims=True)
    idx = jnp.min(jnp.where(scores == m, iota_k, CB_SIZE), axis=0,
                  keepdims=True)
    onehot = (iota_k == idx).astype(jnp.float32)

    quant = jax.lax.dot(
        cb.T.astype(jnp.bfloat16), onehot.astype(jnp.bfloat16),
        preferred_element_type=jnp.float32)
    hist = jnp.sum(onehot, axis=1, keepdims=True).T
    loss = jnp.sum(m)
    return quant, idx, hist, loss


def _vq_kernel(x_ref, cb_ref, quant_ref, idx_ref, hist_ref, loss_ref,
               perp_ref):
    b = pl.program_id(1)
    cb = cb_ref[0]
    cbnorm = jnp.sum(cb * cb, axis=1, keepdims=True)
    iota_k = jax.lax.broadcasted_iota(jnp.int32, (CB_SIZE, 1), 0)

    hist_c = None
    loss_c = None
    for s in range(UNROLL):
        quant, idx, hist, loss = _vq_block(x_ref[s, 0], cb, cbnorm, iota_k)
        quant_ref[s, 0] = quant
        idx_ref[s, 0] = idx
        hist_c = hist if hist_c is None else hist_c + hist
        loss_c = loss if loss_c is None else loss_c + loss

    @pl.when(b == 0)
    def _init():
        hist_ref[0] = hist_c
        loss_ref[0] = jnp.full((1, 128), loss_c, jnp.float32)

    @pl.when(b > 0)
    def _acc():
        hist_ref[0] = hist_ref[0] + hist_c
        loss_ref[0] = loss_ref[0] + loss_c

    @pl.when(b == NB - 1)
    def _finalize():
        hist = hist_ref[0]
        probs = hist * (1.0 / (B * N))
        ent = jnp.sum(probs * jnp.log(probs + 1e-10))
        perp_ref[0] = jnp.full((1, 128), jnp.exp(-ent), jnp.float32)
        loss_ref[0] = loss_ref[0] * ((1.0 + BETA) / (B * N * CB_DIM))


@jax.jit
def kernel(x, codebooks):
    xr = x.reshape(B, NUM_Q, CB_DIM, N)
    quant, idx, hist, loss, perp = pl.pallas_call(
        _vq_kernel,
        grid=(NUM_Q, NB),
        in_specs=[
            pl.BlockSpec((UNROLL, 1, CB_DIM, N), lambda i, b: (b, i, 0, 0)),
            pl.BlockSpec((1, CB_SIZE, CB_DIM), lambda i, b: (i, 0, 0)),
        ],
        out_specs=[
            pl.BlockSpec((UNROLL, 1, CB_DIM, N), lambda i, b: (b, i, 0, 0)),
            pl.BlockSpec((UNROLL, 1, 1, N), lambda i, b: (b, i, 0, 0)),
            pl.BlockSpec((1, 1, CB_SIZE), lambda i, b: (i, 0, 0)),
            pl.BlockSpec((1, 1, 128), lambda i, b: (i, 0, 0)),
            pl.BlockSpec((1, 1, 128), lambda i, b: (i, 0, 0)),
        ],
        out_shape=[
            jax.ShapeDtypeStruct((B, NUM_Q, CB_DIM, N), jnp.float32),
            jax.ShapeDtypeStruct((B, NUM_Q, 1, N), jnp.int32),
            jax.ShapeDtypeStruct((NUM_Q, 1, CB_SIZE), jnp.float32),
            jax.ShapeDtypeStruct((NUM_Q, 1, 128), jnp.float32),
            jax.ShapeDtypeStruct((NUM_Q, 1, 128), jnp.float32),
        ],
    )(xr, codebooks)
    quantized_cat = quant.reshape(B, NUM_Q * CB_DIM, H, W)
    indices_cat = idx.reshape(B, NUM_Q, H, W)
    loss_cat = loss[:, 0, 0]
    perplexity_cat = perp[:, 0, 0]
    return (quantized_cat, indices_cat, loss_cat, perplexity_cat)
